# Initial kernel scaffold; baseline (speedup 1.0000x reference)
import jax
import jax.numpy as jnp
from jax.experimental import pallas as pl


def kernel(data, CT_data, fc1_W, fc1_b, Wh_W, Wh_b, Wt_W, Wt_b, lin1_W, lin1_b, lin2_W, lin2_b, att1_W, att1_b, att2_W, att2_b, norm_g, norm_beta, fc_W, fc_b):
    raise NotImplementedError("write your pallas kernel here")


# speedup vs baseline: 9.4655x; 9.4655x over previous
import functools

import jax
import jax.numpy as jnp
from jax import lax
from jax.experimental import pallas as pl
from jax.experimental.pallas import tpu as pltpu
from jax.experimental.pallas import tpu_sc as plsc

N = 4096
DIN = 384
DH = 512
TK = 6
BR = 256

_NC = 2
_NS = 16
_NW = _NC * _NS
_B = N * TK
_BPW = _B // _NW
_CH = 96
_NCHUNK = _BPW // _CH


def _leaky(x):
    return jnp.where(x >= 0, x, 0.01 * x)


def _k1_body(data_ref, w_ref, b_ref, h1_ref, sum_ref):
    i = pl.program_id(0)
    h = jnp.dot(data_ref[...], w_ref[...], preferred_element_type=jnp.float32)
    h = _leaky(h + b_ref[...])
    h1_ref[...] = h

    @pl.when(i == 0)
    def _():
        sum_ref[...] = jnp.zeros_like(sum_ref)

    sum_ref[...] += jnp.sum(h, axis=0, keepdims=True)


def _k2_body(h1_ref, sum_ref, whw_ref, whb_ref, wtw_ref, wtb_ref,
             eh_ref, et_ref):
    x = (h1_ref[...] + sum_ref[...] * (1.0 / N)) * 0.5
    eh_ref[...] = jnp.dot(x, whw_ref[...], preferred_element_type=jnp.float32) + whb_ref[...]
    et_ref[...] = jnp.dot(x, wtw_ref[...], preferred_element_type=jnp.float32) + wtb_ref[...]


def _k3_body(eh_ref, et_ref, prob_ref, idx_ref):
    scale = DH ** (-0.5)
    logits = lax.dot_general(
        eh_ref[...] * scale, et_ref[...],
        (((1,), (1,)), ((), ())), preferred_element_type=jnp.float32)
    iota = lax.broadcasted_iota(jnp.int32, logits.shape, 1)
    vals, idxs = [], []
    for _ in range(TK):
        m = jnp.max(logits, axis=1, keepdims=True)
        j = jnp.min(jnp.where(logits >= m, iota, N), axis=1, keepdims=True)
        vals.append(m)
        idxs.append(j)
        logits = jnp.where(iota == j, -jnp.inf, logits)
    v = jnp.concatenate(vals, axis=1)
    ji = jnp.concatenate(idxs, axis=1)
    e = jnp.exp(v - v[:, 0:1])
    prob_ref[...] = e / jnp.sum(e, axis=1, keepdims=True)
    idx_ref[...] = ji


def _sc_gather_body(table_hbm, idx_hbm, out_hbm, idx_v, buf_v, sem):
    wid = lax.axis_index("s") * _NC + lax.axis_index("c")
    base = wid * _BPW
    for c in range(_NCHUNK):
        off = base + c * _CH
        pltpu.sync_copy(idx_hbm.at[pl.ds(off, _CH)], idx_v)
        pltpu.async_copy(table_hbm.at[idx_v], buf_v, sem).wait()
        pltpu.sync_copy(buf_v, out_hbm.at[pl.ds(off, _CH)])


def _gather_rows(table, idx):
    mesh = plsc.VectorSubcoreMesh(
        core_axis_name="c", subcore_axis_name="s",
        num_cores=_NC, num_subcores=_NS)
    f = functools.partial(
        pl.kernel, mesh=mesh,
        out_type=jax.ShapeDtypeStruct((_B, DH), jnp.float32),
        scratch_types=[
            pltpu.VMEM((_CH,), jnp.int32),
            pltpu.VMEM((_CH, DH), jnp.float32),
            pltpu.SemaphoreType.DMA,
        ],
    )(_sc_gather_body)
    return f(table, idx)


def _k4_body(nb_ref, eh_ref, p_ref, l1w_ref, l1b_ref, l2w_ref, l2b_ref,
             a1w_ref, a1b_ref, a2w_ref, a2b_ref, emb_ref, g_ref):
    Nb = nb_ref[...]
    eh = eh_ref[...]
    p3 = p_ref[...][:, :, None]
    eh3 = eh[:, None, :]
    eh_r = p3 * Nb + (1.0 - p3) * eh3
    gate = jnp.tanh(eh3 + eh_r)
    ka = jnp.sum(Nb, axis=2) * jnp.sum(gate, axis=2)
    m = jnp.max(ka, axis=1, keepdims=True)
    e = jnp.exp(ka - m)
    kp = e / jnp.sum(e, axis=1, keepdims=True)
    eNh = jnp.sum(kp[:, :, None] * Nb, axis=1)
    s = _leaky(jnp.dot(eh + eNh, l1w_ref[...], preferred_element_type=jnp.float32) + l1b_ref[...])
    bi = _leaky(jnp.dot(eh * eNh, l2w_ref[...], preferred_element_type=jnp.float32) + l2b_ref[...])
    emb = s + bi
    emb_ref[...] = emb
    a1 = _leaky(jnp.dot(emb, a1w_ref[...], preferred_element_type=jnp.float32) + a1b_ref[...])
    g_ref[...] = jnp.dot(a1, a2w_ref[...], preferred_element_type=jnp.float32) + a2b_ref[...]


def _k5_body(emb_ref, g_ref, ng_ref, nbeta_ref, fcw_ref, fcb_ref,
             lg_ref, yp_ref, yh_ref):
    h = emb_ref[...]
    g = g_ref[...]
    m = jnp.max(g, axis=0, keepdims=True)
    e = jnp.exp(g - m)
    a = e / jnp.sum(e, axis=0, keepdims=True)
    hr = jnp.sum(a * h, axis=0, keepdims=True)
    mu = jnp.mean(hr, axis=1, keepdims=True)
    var = jnp.mean((hr - mu) ** 2, axis=1, keepdims=True)
    hn = (hr - mu) / jnp.sqrt(var + 1e-5) * ng_ref[...] + nbeta_ref[...]
    lg = jnp.dot(hn, fcw_ref[...], preferred_element_type=jnp.float32) + fcb_ref[...]
    lg_ref[...] = lg
    mm = jnp.max(lg, axis=1, keepdims=True)
    ee = jnp.exp(lg - mm)
    yp_ref[...] = ee / jnp.sum(ee, axis=1, keepdims=True)
    yh_ref[...] = jnp.where(lg[:, 1:2] > lg[:, 0:1], 1, 0).astype(jnp.int32)


def kernel(data, CT_data, fc1_W, fc1_b, Wh_W, Wh_b, Wt_W, Wt_b,
           lin1_W, lin1_b, lin2_W, lin2_b, att1_W, att1_b, att2_W, att2_b,
           norm_g, norm_beta, fc_W, fc_b):
    del CT_data
    x0 = jnp.squeeze(data, axis=0)
    r2 = lambda v: v.reshape(1, -1)
    ngrid = N // BR
    full = lambda a, b: pl.BlockSpec((a, b), lambda i: (0, 0))
    rows = lambda b: pl.BlockSpec((BR, b), lambda i: (i, 0))

    h1, csum = pl.pallas_call(
        _k1_body,
        grid=(ngrid,),
        in_specs=[rows(DIN), full(DIN, DH), full(1, DH)],
        out_specs=[rows(DH), full(1, DH)],
        out_shape=[jax.ShapeDtypeStruct((N, DH), jnp.float32),
                   jax.ShapeDtypeStruct((1, DH), jnp.float32)],
    )(x0, fc1_W, r2(fc1_b))

    e_h, e_t = pl.pallas_call(
        _k2_body,
        grid=(ngrid,),
        in_specs=[rows(DH), full(1, DH), full(DH, DH), full(1, DH),
                  full(DH, DH), full(1, DH)],
        out_specs=[rows(DH), rows(DH)],
        out_shape=[jax.ShapeDtypeStruct((N, DH), jnp.float32),
                   jax.ShapeDtypeStruct((N, DH), jnp.float32)],
    )(h1, csum, Wh_W, r2(Wh_b), Wt_W, r2(Wt_b))

    probs, idx = pl.pallas_call(
        _k3_body,
        grid=(ngrid,),
        in_specs=[rows(DH), full(N, DH)],
        out_specs=[rows(TK), rows(TK)],
        out_shape=[jax.ShapeDtypeStruct((N, TK), jnp.float32),
                   jax.ShapeDtypeStruct((N, TK), jnp.int32)],
    )(e_h, e_t)

    nb = _gather_rows(e_t, idx.reshape(_B))
    nb3 = nb.reshape(N, TK, DH)

    emb, gsc = pl.pallas_call(
        _k4_body,
        grid=(ngrid,),
        in_specs=[pl.BlockSpec((BR, TK, DH), lambda i: (i, 0, 0)),
                  rows(DH), rows(TK),
                  full(DH, DH), full(1, DH), full(DH, DH), full(1, DH),
                  full(DH, DH // 2), full(1, DH // 2), full(DH // 2, 1),
                  full(1, 1)],
        out_specs=[rows(DH), rows(1)],
        out_shape=[jax.ShapeDtypeStruct((N, DH), jnp.float32),
                   jax.ShapeDtypeStruct((N, 1), jnp.float32)],
    )(nb3, e_h, probs, lin1_W, r2(lin1_b), lin2_W, r2(lin2_b),
      att1_W, r2(att1_b), att2_W, r2(att2_b))

    logits, y_prob, y_hat = pl.pallas_call(
        _k5_body,
        in_specs=[pl.BlockSpec((N, DH), lambda: (0, 0)),
                  pl.BlockSpec((N, 1), lambda: (0, 0)),
                  pl.BlockSpec((1, DH), lambda: (0, 0)),
                  pl.BlockSpec((1, DH), lambda: (0, 0)),
                  pl.BlockSpec((DH, 2), lambda: (0, 0)),
                  pl.BlockSpec((1, 2), lambda: (0, 0))],
        out_specs=[pl.BlockSpec((1, 2), lambda: (0, 0)),
                   pl.BlockSpec((1, 2), lambda: (0, 0)),
                   pl.BlockSpec((1, 1), lambda: (0, 0))],
        out_shape=[jax.ShapeDtypeStruct((1, 2), jnp.float32),
                   jax.ShapeDtypeStruct((1, 2), jnp.float32),
                   jax.ShapeDtypeStruct((1, 1), jnp.int32)],
    )(emb, gsc, r2(norm_g), r2(norm_beta), fc_W, r2(fc_b))

    return (logits, y_prob, y_hat)


# speedup vs baseline: 9.5078x; 1.0045x over previous
import functools

import jax
import jax.numpy as jnp
from jax import lax
from jax.experimental import pallas as pl
from jax.experimental.pallas import tpu as pltpu
from jax.experimental.pallas import tpu_sc as plsc

N = 4096
DIN = 384
DH = 512
TK = 6
BR = 256

_NC = 2
_NS = 16
_NW = _NC * _NS
_B = N * TK
_BPW = _B // _NW
_CH = 96
_NCHUNK = _BPW // _CH


def _leaky(x):
    return jnp.where(x >= 0, x, 0.01 * x)


def _k1_body(data_ref, w_ref, b_ref, h1_ref, sum_ref):
    i = pl.program_id(0)
    h = jnp.dot(data_ref[...], w_ref[...], preferred_element_type=jnp.float32)
    h = _leaky(h + b_ref[...])
    h1_ref[...] = h

    @pl.when(i == 0)
    def _():
        sum_ref[...] = jnp.zeros_like(sum_ref)

    sum_ref[...] += jnp.sum(h, axis=0, keepdims=True)


def _k2_body(h1_ref, sum_ref, whw_ref, whb_ref, wtw_ref, wtb_ref,
             eh_ref, et_ref):
    x = (h1_ref[...] + sum_ref[...] * (1.0 / N)) * 0.5
    eh_ref[...] = jnp.dot(x, whw_ref[...], preferred_element_type=jnp.float32) + whb_ref[...]
    et_ref[...] = jnp.dot(x, wtw_ref[...], preferred_element_type=jnp.float32) + wtb_ref[...]


def _k3_body(eh_ref, et_ref, prob_ref, idx_ref):
    scale = DH ** (-0.5)
    logits = lax.dot_general(
        eh_ref[...] * scale, et_ref[...],
        (((1,), (1,)), ((), ())), preferred_element_type=jnp.float32)
    iota = lax.broadcasted_iota(jnp.int32, logits.shape, 1)
    vals, idxs = [], []
    for _ in range(TK):
        m = jnp.max(logits, axis=1, keepdims=True)
        j = jnp.min(jnp.where(logits >= m, iota, N), axis=1, keepdims=True)
        vals.append(m)
        idxs.append(j)
        logits = jnp.where(iota == j, -jnp.inf, logits)
    v = jnp.concatenate(vals, axis=1)
    ji = jnp.concatenate(idxs, axis=1)
    e = jnp.exp(v - v[:, 0:1])
    prob_ref[...] = e / jnp.sum(e, axis=1, keepdims=True)
    idx_ref[...] = ji


def _sc_gather_body(table_hbm, idx_hbm, out_hbm, idx_v, b0, b1,
                    sg0, sg1, ss0, ss1):
    wid = lax.axis_index("s") * _NC + lax.axis_index("c")
    base = wid * _BPW
    pltpu.sync_copy(idx_hbm.at[pl.ds(base, _BPW)], idx_v)
    bufs, gsem, ssem = (b0, b1), (sg0, sg1), (ss0, ss1)
    gh = [None, None]
    sh = [None, None]
    gh[0] = pltpu.async_copy(table_hbm.at[idx_v.at[pl.ds(0, _CH)]], b0, sg0)
    for c in range(_NCHUNK):
        cur = c & 1
        nxt = 1 - cur
        if c + 1 < _NCHUNK:
            if sh[nxt] is not None:
                sh[nxt].wait()
            gh[nxt] = pltpu.async_copy(
                table_hbm.at[idx_v.at[pl.ds((c + 1) * _CH, _CH)]],
                bufs[nxt], gsem[nxt])
        gh[cur].wait()
        sh[cur] = pltpu.async_copy(
            bufs[cur], out_hbm.at[pl.ds(base + c * _CH, _CH)], ssem[cur])
    sh[0].wait()
    sh[1].wait()


def _gather_rows(table, idx):
    mesh = plsc.VectorSubcoreMesh(
        core_axis_name="c", subcore_axis_name="s",
        num_cores=_NC, num_subcores=_NS)
    f = functools.partial(
        pl.kernel, mesh=mesh,
        out_type=jax.ShapeDtypeStruct((_B, DH), jnp.float32),
        scratch_types=[
            pltpu.VMEM((_BPW,), jnp.int32),
            pltpu.VMEM((_CH, DH), jnp.float32),
            pltpu.VMEM((_CH, DH), jnp.float32),
            pltpu.SemaphoreType.DMA,
            pltpu.SemaphoreType.DMA,
            pltpu.SemaphoreType.DMA,
            pltpu.SemaphoreType.DMA,
        ],
    )(_sc_gather_body)
    return f(table, idx)


def _k4_body(nb_ref, eh_ref, p_ref, l1w_ref, l1b_ref, l2w_ref, l2b_ref,
             a1w_ref, a1b_ref, a2w_ref, a2b_ref, emb_ref, g_ref):
    Nb = nb_ref[...]
    eh = eh_ref[...]
    p3 = p_ref[...][:, :, None]
    eh3 = eh[:, None, :]
    eh_r = p3 * Nb + (1.0 - p3) * eh3
    gate = jnp.tanh(eh3 + eh_r)
    ka = jnp.sum(Nb, axis=2) * jnp.sum(gate, axis=2)
    m = jnp.max(ka, axis=1, keepdims=True)
    e = jnp.exp(ka - m)
    kp = e / jnp.sum(e, axis=1, keepdims=True)
    eNh = jnp.sum(kp[:, :, None] * Nb, axis=1)
    s = _leaky(jnp.dot(eh + eNh, l1w_ref[...], preferred_element_type=jnp.float32) + l1b_ref[...])
    bi = _leaky(jnp.dot(eh * eNh, l2w_ref[...], preferred_element_type=jnp.float32) + l2b_ref[...])
    emb = s + bi
    emb_ref[...] = emb
    a1 = _leaky(jnp.dot(emb, a1w_ref[...], preferred_element_type=jnp.float32) + a1b_ref[...])
    g_ref[...] = jnp.dot(a1, a2w_ref[...], preferred_element_type=jnp.float32) + a2b_ref[...]


def _k5_body(emb_ref, g_ref, ng_ref, nbeta_ref, fcw_ref, fcb_ref,
             lg_ref, yp_ref, yh_ref):
    h = emb_ref[...]
    g = g_ref[...]
    m = jnp.max(g, axis=0, keepdims=True)
    e = jnp.exp(g - m)
    a = e / jnp.sum(e, axis=0, keepdims=True)
    hr = jnp.sum(a * h, axis=0, keepdims=True)
    mu = jnp.mean(hr, axis=1, keepdims=True)
    var = jnp.mean((hr - mu) ** 2, axis=1, keepdims=True)
    hn = (hr - mu) / jnp.sqrt(var + 1e-5) * ng_ref[...] + nbeta_ref[...]
    lg = jnp.dot(hn, fcw_ref[...], preferred_element_type=jnp.float32) + fcb_ref[...]
    lg_ref[...] = lg
    mm = jnp.max(lg, axis=1, keepdims=True)
    ee = jnp.exp(lg - mm)
    yp_ref[...] = ee / jnp.sum(ee, axis=1, keepdims=True)
    yh_ref[...] = jnp.where(lg[:, 1:2] > lg[:, 0:1], 1, 0).astype(jnp.int32)


def kernel(data, CT_data, fc1_W, fc1_b, Wh_W, Wh_b, Wt_W, Wt_b,
           lin1_W, lin1_b, lin2_W, lin2_b, att1_W, att1_b, att2_W, att2_b,
           norm_g, norm_beta, fc_W, fc_b):
    del CT_data
    x0 = jnp.squeeze(data, axis=0)
    r2 = lambda v: v.reshape(1, -1)
    ngrid = N // BR
    full = lambda a, b: pl.BlockSpec((a, b), lambda i: (0, 0))
    rows = lambda b: pl.BlockSpec((BR, b), lambda i: (i, 0))

    h1, csum = pl.pallas_call(
        _k1_body,
        grid=(ngrid,),
        in_specs=[rows(DIN), full(DIN, DH), full(1, DH)],
        out_specs=[rows(DH), full(1, DH)],
        out_shape=[jax.ShapeDtypeStruct((N, DH), jnp.float32),
                   jax.ShapeDtypeStruct((1, DH), jnp.float32)],
    )(x0, fc1_W, r2(fc1_b))

    e_h, e_t = pl.pallas_call(
        _k2_body,
        grid=(ngrid,),
        in_specs=[rows(DH), full(1, DH), full(DH, DH), full(1, DH),
                  full(DH, DH), full(1, DH)],
        out_specs=[rows(DH), rows(DH)],
        out_shape=[jax.ShapeDtypeStruct((N, DH), jnp.float32),
                   jax.ShapeDtypeStruct((N, DH), jnp.float32)],
    )(h1, csum, Wh_W, r2(Wh_b), Wt_W, r2(Wt_b))

    probs, idx = pl.pallas_call(
        _k3_body,
        grid=(ngrid,),
        in_specs=[rows(DH), full(N, DH)],
        out_specs=[rows(TK), rows(TK)],
        out_shape=[jax.ShapeDtypeStruct((N, TK), jnp.float32),
                   jax.ShapeDtypeStruct((N, TK), jnp.int32)],
    )(e_h, e_t)

    nb = _gather_rows(e_t, idx.reshape(_B))
    nb3 = nb.reshape(N, TK, DH)

    emb, gsc = pl.pallas_call(
        _k4_body,
        grid=(ngrid,),
        in_specs=[pl.BlockSpec((BR, TK, DH), lambda i: (i, 0, 0)),
                  rows(DH), rows(TK),
                  full(DH, DH), full(1, DH), full(DH, DH), full(1, DH),
                  full(DH, DH // 2), full(1, DH // 2), full(DH // 2, 1),
                  full(1, 1)],
        out_specs=[rows(DH), rows(1)],
        out_shape=[jax.ShapeDtypeStruct((N, DH), jnp.float32),
                   jax.ShapeDtypeStruct((N, 1), jnp.float32)],
    )(nb3, e_h, probs, lin1_W, r2(lin1_b), lin2_W, r2(lin2_b),
      att1_W, r2(att1_b), att2_W, r2(att2_b))

    logits, y_prob, y_hat = pl.pallas_call(
        _k5_body,
        in_specs=[pl.BlockSpec((N, DH), lambda: (0, 0)),
                  pl.BlockSpec((N, 1), lambda: (0, 0)),
                  pl.BlockSpec((1, DH), lambda: (0, 0)),
                  pl.BlockSpec((1, DH), lambda: (0, 0)),
                  pl.BlockSpec((DH, 2), lambda: (0, 0)),
                  pl.BlockSpec((1, 2), lambda: (0, 0))],
        out_specs=[pl.BlockSpec((1, 2), lambda: (0, 0)),
                   pl.BlockSpec((1, 2), lambda: (0, 0)),
                   pl.BlockSpec((1, 1), lambda: (0, 0))],
        out_shape=[jax.ShapeDtypeStruct((1, 2), jnp.float32),
                   jax.ShapeDtypeStruct((1, 2), jnp.float32),
                   jax.ShapeDtypeStruct((1, 1), jnp.int32)],
    )(emb, gsc, r2(norm_g), r2(norm_beta), fc_W, r2(fc_b))

    return (logits, y_prob, y_hat)


# speedup vs baseline: 9.7829x; 1.0289x over previous
import functools

import jax
import jax.numpy as jnp
from jax import lax
from jax.experimental import pallas as pl
from jax.experimental.pallas import tpu as pltpu
from jax.experimental.pallas import tpu_sc as plsc

N = 4096
DIN = 384
DH = 512
TK = 6
BR = 256
NB_BLK = N // BR

_NC = 2
_NS = 16
_NW = _NC * _NS
_B = N * TK
_BPW = _B // _NW
_CH = 96
_NCHUNK = _BPW // _CH


def _leaky(x):
    return jnp.where(x >= 0, x, 0.01 * x)


def _dot(a, b):
    return jnp.dot(a, b, preferred_element_type=jnp.float32)


def _ka_body(data_ref, fc1w_ref, fc1b_ref, whw_ref, whb_ref, wtw_ref, wtb_ref,
             eh_ref, et_ref, prob_ref, idx_ref,
             h1_s, eh_s, et_s, csum_s):
    i = pl.program_id(0)

    @pl.when(i < NB_BLK)
    def _phase0():
        h = _leaky(_dot(data_ref[...], fc1w_ref[...]) + fc1b_ref[...])
        h1_s[pl.ds(i * BR, BR), :] = h

        @pl.when(i == 0)
        def _():
            csum_s[...] = jnp.zeros_like(csum_s)

        csum_s[...] += jnp.sum(h, axis=0, keepdims=True)

    @pl.when(jnp.logical_and(i >= NB_BLK, i < 2 * NB_BLK))
    def _phase1():
        j = i - NB_BLK
        x = (h1_s[pl.ds(j * BR, BR), :] + csum_s[...] * (1.0 / N)) * 0.5
        eh_s[pl.ds(j * BR, BR), :] = _dot(x, whw_ref[...]) + whb_ref[...]
        et_s[pl.ds(j * BR, BR), :] = _dot(x, wtw_ref[...]) + wtb_ref[...]

    @pl.when(i >= 2 * NB_BLK)
    def _phase2():
        j = i - 2 * NB_BLK
        eh = eh_s[pl.ds(j * BR, BR), :]
        et = et_s[pl.ds(j * BR, BR), :]
        scale = DH ** (-0.5)
        logits = lax.dot_general(eh * scale, et_s[...],
                                 (((1,), (1,)), ((), ())),
                                 preferred_element_type=jnp.float32)
        iota = lax.broadcasted_iota(jnp.int32, logits.shape, 1)
        vals, idxs = [], []
        for _ in range(TK):
            m = jnp.max(logits, axis=1, keepdims=True)
            jj = jnp.min(jnp.where(logits >= m, iota, N), axis=1, keepdims=True)
            vals.append(m)
            idxs.append(jj)
            logits = jnp.where(iota == jj, -jnp.inf, logits)
        v = jnp.concatenate(vals, axis=1)
        ji = jnp.concatenate(idxs, axis=1)
        e = jnp.exp(v - v[:, 0:1])
        prob_ref[...] = e / jnp.sum(e, axis=1, keepdims=True)
        idx_ref[...] = ji
        eh_ref[...] = eh
        et_ref[...] = et


def _sc_gather_body(table_hbm, idx_hbm, out_hbm, idx_v, b0, b1,
                    sg0, sg1, ss0, ss1):
    wid = lax.axis_index("s") * _NC + lax.axis_index("c")
    base = wid * _BPW
    pltpu.sync_copy(idx_hbm.at[pl.ds(base, _BPW)], idx_v)
    bufs, gsem, ssem = (b0, b1), (sg0, sg1), (ss0, ss1)
    gh = [None, None]
    sh = [None, None]
    gh[0] = pltpu.async_copy(table_hbm.at[idx_v.at[pl.ds(0, _CH)]], b0, sg0)
    for c in range(_NCHUNK):
        cur = c & 1
        nxt = 1 - cur
        if c + 1 < _NCHUNK:
            if sh[nxt] is not None:
                sh[nxt].wait()
            gh[nxt] = pltpu.async_copy(
                table_hbm.at[idx_v.at[pl.ds((c + 1) * _CH, _CH)]],
                bufs[nxt], gsem[nxt])
        gh[cur].wait()
        sh[cur] = pltpu.async_copy(
            bufs[cur], out_hbm.at[pl.ds(base + c * _CH, _CH)], ssem[cur])
    sh[0].wait()
    sh[1].wait()


def _gather_rows(table, idx):
    mesh = plsc.VectorSubcoreMesh(
        core_axis_name="c", subcore_axis_name="s",
        num_cores=_NC, num_subcores=_NS)
    f = functools.partial(
        pl.kernel, mesh=mesh,
        out_type=jax.ShapeDtypeStruct((_B, DH), jnp.float32),
        scratch_types=[
            pltpu.VMEM((_BPW,), jnp.int32),
            pltpu.VMEM((_CH, DH), jnp.float32),
            pltpu.VMEM((_CH, DH), jnp.float32),
            pltpu.SemaphoreType.DMA,
            pltpu.SemaphoreType.DMA,
            pltpu.SemaphoreType.DMA,
            pltpu.SemaphoreType.DMA,
        ],
    )(_sc_gather_body)
    return f(table, idx)


def _kc_body(nb_ref, eh_ref, p_ref, l1w_ref, l1b_ref, l2w_ref, l2b_ref,
             a1w_ref, a1b_ref, a2w_ref, a2b_ref,
             ng_ref, nbeta_ref, fcw_ref, fcb_ref,
             lg_ref, yp_ref, yh_ref, emb_s, g_s):
    i = pl.program_id(0)

    @pl.when(i < NB_BLK)
    def _combine():
        Nb = nb_ref[...]
        eh = eh_ref[...]
        p3 = p_ref[...][:, :, None]
        eh3 = eh[:, None, :]
        eh_r = p3 * Nb + (1.0 - p3) * eh3
        gate = jnp.tanh(eh3 + eh_r)
        ka = jnp.sum(Nb, axis=2) * jnp.sum(gate, axis=2)
        m = jnp.max(ka, axis=1, keepdims=True)
        e = jnp.exp(ka - m)
        kp = e / jnp.sum(e, axis=1, keepdims=True)
        eNh = jnp.sum(kp[:, :, None] * Nb, axis=1)
        s = _leaky(_dot(eh + eNh, l1w_ref[...]) + l1b_ref[...])
        bi = _leaky(_dot(eh * eNh, l2w_ref[...]) + l2b_ref[...])
        emb = s + bi
        emb_s[pl.ds(i * BR, BR), :] = emb
        a1 = _leaky(_dot(emb, a1w_ref[...]) + a1b_ref[...])
        g_s[pl.ds(i * BR, BR), :] = _dot(a1, a2w_ref[...]) + a2b_ref[...]

    @pl.when(i == NB_BLK)
    def _readout():
        h = emb_s[...]
        g = g_s[...]
        m = jnp.max(g, axis=0, keepdims=True)
        e = jnp.exp(g - m)
        a = e / jnp.sum(e, axis=0, keepdims=True)
        hr = jnp.sum(a * h, axis=0, keepdims=True)
        mu = jnp.mean(hr, axis=1, keepdims=True)
        var = jnp.mean((hr - mu) ** 2, axis=1, keepdims=True)
        hn = (hr - mu) / jnp.sqrt(var + 1e-5) * ng_ref[...] + nbeta_ref[...]
        lg = _dot(hn, fcw_ref[...]) + fcb_ref[...]
        lg_ref[...] = lg
        mm = jnp.max(lg, axis=1, keepdims=True)
        ee = jnp.exp(lg - mm)
        yp_ref[...] = ee / jnp.sum(ee, axis=1, keepdims=True)
        yh_ref[...] = jnp.where(lg[:, 1:2] > lg[:, 0:1], 1, 0).astype(jnp.int32)


def kernel(data, CT_data, fc1_W, fc1_b, Wh_W, Wh_b, Wt_W, Wt_b,
           lin1_W, lin1_b, lin2_W, lin2_b, att1_W, att1_b, att2_W, att2_b,
           norm_g, norm_beta, fc_W, fc_b):
    del CT_data
    x0 = jnp.squeeze(data, axis=0)
    r2 = lambda v: v.reshape(1, -1)
    full = lambda a, b: pl.BlockSpec((a, b), lambda i: (0, 0))
    p2rows = lambda b: pl.BlockSpec(
        (BR, b), lambda i: (jnp.where(i < 2 * NB_BLK, 0, i - 2 * NB_BLK), 0))

    e_h, e_t, probs, idx = pl.pallas_call(
        _ka_body,
        grid=(3 * NB_BLK,),
        in_specs=[pl.BlockSpec((BR, DIN), lambda i: (jnp.minimum(i, NB_BLK - 1), 0)),
                  full(DIN, DH), full(1, DH),
                  full(DH, DH), full(1, DH),
                  full(DH, DH), full(1, DH)],
        out_specs=[p2rows(DH), p2rows(DH), p2rows(TK), p2rows(TK)],
        out_shape=[jax.ShapeDtypeStruct((N, DH), jnp.float32),
                   jax.ShapeDtypeStruct((N, DH), jnp.float32),
                   jax.ShapeDtypeStruct((N, TK), jnp.float32),
                   jax.ShapeDtypeStruct((N, TK), jnp.int32)],
        scratch_shapes=[pltpu.VMEM((N, DH), jnp.float32),
                        pltpu.VMEM((N, DH), jnp.float32),
                        pltpu.VMEM((N, DH), jnp.float32),
                        pltpu.VMEM((1, DH), jnp.float32)],
    )(x0, fc1_W, r2(fc1_b), Wh_W, r2(Wh_b), Wt_W, r2(Wt_b))

    nb = _gather_rows(e_t, idx.reshape(_B))
    nb3 = nb.reshape(N, TK, DH)

    rows16 = lambda b: pl.BlockSpec((BR, b), lambda i: (jnp.minimum(i, NB_BLK - 1), 0))
    out01 = lambda a, b: pl.BlockSpec((a, b), lambda i: (0, 0))

    logits, y_prob, y_hat = pl.pallas_call(
        _kc_body,
        grid=(NB_BLK + 1,),
        in_specs=[pl.BlockSpec((BR, TK, DH),
                               lambda i: (jnp.minimum(i, NB_BLK - 1), 0, 0)),
                  rows16(DH), rows16(TK),
                  full(DH, DH), full(1, DH), full(DH, DH), full(1, DH),
                  full(DH, DH // 2), full(1, DH // 2), full(DH // 2, 1),
                  full(1, 1),
                  full(1, DH), full(1, DH), full(DH, 2), full(1, 2)],
        out_specs=[out01(1, 2), out01(1, 2), out01(1, 1)],
        out_shape=[jax.ShapeDtypeStruct((1, 2), jnp.float32),
                   jax.ShapeDtypeStruct((1, 2), jnp.float32),
                   jax.ShapeDtypeStruct((1, 1), jnp.int32)],
        scratch_shapes=[pltpu.VMEM((N, DH), jnp.float32),
                        pltpu.VMEM((N, 1), jnp.float32)],
    )(nb3, e_h, probs, lin1_W, r2(lin1_b), lin2_W, r2(lin2_b),
      att1_W, r2(att1_b), att2_W, r2(att2_b),
      r2(norm_g), r2(norm_beta), fc_W, r2(fc_b))

    return (logits, y_prob, y_hat)
